# Initial kernel scaffold; baseline (speedup 1.0000x reference)
import jax
import jax.numpy as jnp
from jax.experimental import pallas as pl


def kernel(text_embeds, image_patch_embedding, bbox, attention_mask):
    raise NotImplementedError("write your pallas kernel here")


# speedup vs baseline: 4.5018x; 4.5018x over previous
import functools

import jax
import jax.numpy as jnp
from jax.experimental import pallas as pl
from jax.experimental.pallas import tpu as pltpu

B, L, C, P = 64, 512, 768, 196


def _aet_body(bbox_ref, mask_ref, text_ref, image_ref, out_ref):
    d = bbox_ref[0] // 72
    x0 = d[0:1, :]
    y0 = d[1:2, :]
    x1 = d[2:3, :]
    y1 = d[3:4, :]
    valid = (x0 == x1) & (y0 == y1) & (mask_ref[0] != 0)
    lab = jnp.where(valid, y0 * 14 + x0, -100)

    rows = jax.lax.broadcasted_iota(jnp.int32, (P, L), 0)
    oh = (rows == lab).astype(jnp.float32)
    sums = jnp.dot(oh, text_ref[0], preferred_element_type=jnp.float32)
    cnts = jnp.sum(oh, axis=1, keepdims=True)
    tpe = sums * (1.0 / jnp.maximum(cnts, 1.0))

    m = jax.lax.dot_general(
        image_ref[0], tpe, (((1,), (1,)), ((), ())),
        preferred_element_type=jnp.float32)

    mr = jnp.max(m, axis=1, keepdims=True)
    lse_r = jnp.log(jnp.sum(jnp.exp(m - mr), axis=1, keepdims=True)) + mr
    mc = jnp.max(m, axis=0, keepdims=True)
    lse_c = jnp.log(jnp.sum(jnp.exp(m - mc), axis=0, keepdims=True)) + mc

    ii = jax.lax.broadcasted_iota(jnp.int32, (P, P), 0)
    jj = jax.lax.broadcasted_iota(jnp.int32, (P, P), 1)
    diag_sum = jnp.sum(jnp.where(ii == jj, m, 0.0))

    loss = ((jnp.sum(lse_r) + jnp.sum(lse_c)) * 0.5 - diag_sum) * (1.0 / P)
    out_ref[...] = loss.reshape(1, 1, 1)


@functools.partial(jax.jit, static_argnames=())
def kernel(text_embeds, image_patch_embedding, bbox, attention_mask):
    bbox_t = jnp.transpose(bbox.astype(jnp.int32), (0, 2, 1))
    mask3 = attention_mask.astype(jnp.int32).reshape(B, 1, L)

    per_sample = pl.pallas_call(
        _aet_body,
        out_shape=jax.ShapeDtypeStruct((B, 1, 1), jnp.float32),
        grid=(B,),
        in_specs=[
            pl.BlockSpec((1, 4, L), lambda b: (b, 0, 0)),
            pl.BlockSpec((1, 1, L), lambda b: (b, 0, 0)),
            pl.BlockSpec((1, L, C), lambda b: (b, 0, 0)),
            pl.BlockSpec((1, P, C), lambda b: (b, 0, 0)),
        ],
        out_specs=pl.BlockSpec((1, 1, 1), lambda b: (b, 0, 0)),
        compiler_params=pltpu.CompilerParams(
            dimension_semantics=("parallel",),
        ),
        name="aet_loss",
    )(bbox_t, mask3, text_embeds, image_patch_embedding)

    return jnp.mean(per_sample)
